# Initial kernel scaffold; baseline (speedup 1.0000x reference)
import jax
import jax.numpy as jnp
from jax.experimental import pallas as pl


def kernel(vertices, faces, in_theta, in_phi, in_freq, coor_embed, angle_embed, area_embed, emnoangle_embed, emangle_embed, emfreq_embed, normal_embed, W, bias):
    raise NotImplementedError("write your pallas kernel here")


# speedup vs baseline: 11.4545x; 11.4545x over previous
import functools
import math

import jax
import jax.numpy as jnp
from jax import lax
from jax.experimental import pallas as pl
from jax.experimental.pallas import tpu as pltpu
from jax.experimental.pallas import tpu_sc as plsc

B, NV, NF = 4, 25000, 50000
ND, DE, DIM, NS = 128, 16, 128, 21
NF_PAD = 51200
G4 = NF_PAD // 256
PI = float(math.pi)

_N_TILE = 6272
_EPI_LAST = 976


def _wid():
    return lax.axis_index("s") * 2 + lax.axis_index("c")


def _tile_range(wid):
    b = wid // 8
    j = wid % 8
    return b, j, j * _N_TILE


def _k0_body(t_ref, w_ref, bias_ref, m_ref):
    for s in range(NS):
        prod = jnp.dot(t_ref[s], w_ref[s], preferred_element_type=jnp.float32)
        if s == 0:
            prod = prod + bias_ref[0][None, :]
        m_ref[s * ND:(s + 1) * ND, :] = prod


def _fold_tables(tables, w_r, bias2d):
    return pl.pallas_call(
        _k0_body,
        out_shape=jax.ShapeDtypeStruct((NS * ND, DIM), jnp.float32),
    )(tables, w_r, bias2d)


def _k1_body(verts_hbm, faces_hbm, fc_hbm, verts_v, fbuf, cbuf):
    wid = _wid()
    b, j, rel0 = _tile_range(wid)
    pltpu.sync_copy(verts_hbm.at[b], verts_v)

    def gather_group(g, carry):
        i16 = (lax.iota(jnp.int32, 16) + g * 16) * 3
        for p in range(3):
            vid = plsc.load_gather(fbuf, [i16 + p]) * 3
            for c in range(3):
                val = plsc.load_gather(verts_v, [vid + c])
                cbuf[3 * p + c, pl.ds(g * 16, 16)] = val
        return carry

    def chunk(c, carry):
        off = rel0 + c * 1024
        pltpu.sync_copy(faces_hbm.at[b, pl.ds(off * 3, 3072)], fbuf)
        lax.fori_loop(0, 64, gather_group, 0)
        pltpu.sync_copy(cbuf, fc_hbm.at[b, :, pl.ds(off, 1024)])
        return carry

    n_ch = jnp.where(j < 7, 6, 5)
    lax.fori_loop(0, n_ch, chunk, 0)

    off_e = rel0 + n_ch * 1024
    pltpu.sync_copy(faces_hbm.at[b, pl.ds(off_e * 3, 3072)], fbuf)
    n_g = jnp.where(j < 7, 8, _EPI_LAST // 16)
    lax.fori_loop(0, n_g, gather_group, 0)

    @pl.when(j < 7)
    def _():
        pltpu.sync_copy(cbuf.at[:, pl.ds(0, 128)],
                        fc_hbm.at[b, :, pl.ds(off_e, 128)])

    @pl.when(j >= 7)
    def _():
        pltpu.sync_copy(cbuf, fc_hbm.at[b, :, pl.ds(off_e, 1024)])


def _gather_face_coords(vertices, faces):
    mesh = plsc.VectorSubcoreMesh(core_axis_name="c", subcore_axis_name="s")
    return pl.kernel(
        _k1_body,
        out_type=jax.ShapeDtypeStruct((B, 9, NF_PAD), jnp.float32),
        mesh=mesh,
        compiler_params=pltpu.CompilerParams(needs_layout_passes=False),
        scratch_types=[
            pltpu.VMEM((NV * 3,), jnp.float32),
            pltpu.VMEM((3072,), jnp.int32),
            pltpu.VMEM((9, 1024), jnp.float32),
        ],
    )(vertices, faces)


def _disc(t, lo, hi):
    u = (t - lo) / (hi - lo) * 128.0 - 0.5
    r = (u + 12582912.0) - 12582912.0
    r = jnp.clip(r, 0.0, 127.0)
    return jnp.clip(r.astype(jnp.int32), 0, 127)


def _acos(x):
    ax = jnp.abs(x)
    p = -0.0012624911
    for c in (0.0066700901, -0.0170881256, 0.0308918810, -0.0501743046,
              0.0889789874, -0.2145988016, 1.5707963050):
        p = p * ax + c
    r = jnp.sqrt(1.0 - ax) * p
    return jnp.where(x >= 0.0, r, PI - r)


def _k2_body(fc_ref, th_ref, ph_ref, fr_ref, out_ref):
    b = pl.program_id(0)
    fc = fc_ref[0]
    x = [fc[r] for r in range(9)]
    sh = [x[6], x[7], x[8], x[0], x[1], x[2], x[3], x[4], x[5]]
    e = [x[r] - sh[r] for r in range(9)]
    nv = []
    for p in range(3):
        s = e[3 * p] * e[3 * p] + e[3 * p + 1] * e[3 * p + 1] \
            + e[3 * p + 2] * e[3 * p + 2]
        inv = 1.0 / (jnp.sqrt(s) + 1e-12)
        nv += [e[3 * p + c] * inv for c in range(3)]
    ang = []
    for c in range(3):
        c2 = (c + 2) % 3
        nd = -(nv[c] * nv[c2] + nv[3 + c] * nv[3 + c2] + nv[6 + c] * nv[6 + c2])
        nd = jnp.clip(nd, -1.0 + 1e-5, 1.0 - 1e-5)
        ang.append(_acos(nd))
    crx = e[1] * e[5] - e[2] * e[4]
    cry = e[2] * e[3] - e[0] * e[5]
    crz = e[0] * e[4] - e[1] * e[3]
    crn = jnp.sqrt(crx * crx + cry * cry + crz * crz)
    inv = 1.0 / (crn + 1e-12)
    nx, ny, nz = crx * inv, cry * inv, crz * inv
    area = crn * 0.5
    th = th_ref[b, 0] * (PI / 180.0)
    ph = ph_ref[b, 0] * (PI / 180.0)
    shp = jnp.sin(ph)
    ivx, ivy, ivz = shp * jnp.cos(th), shp * jnp.sin(th), jnp.cos(ph)
    ivn = jnp.sqrt(ivx * ivx + ivy * ivy + ivz * ivz) + 1e-12
    jvx, jvy, jvz = ivx / ivn, ivy / ivn, ivz / ivn
    nn = jnp.sqrt(nx * nx + ny * ny + nz * nz) + 1e-12
    nd2 = -((nx / nn) * jvx + (ny / nn) * jvy + (nz / nn) * jvz)
    nd2 = jnp.clip(nd2, -1.0 + 1e-5, 1.0 - 1e-5)
    emno = _acos(nd2)
    fr = fr_ref[b, 0]
    ones = jnp.ones_like(x[0])

    slots = (
        [_disc(x[r], 0.0, 1.0) for r in range(9)]
        + [_disc(a, 0.0, PI) for a in ang]
        + [_disc(area, 0.0, 2.0), _disc(emno, 0.0, PI)]
        + [_disc(iv * ones, -1.0, 1.0) for iv in (ivx, ivy, ivz)]
        + [_disc(fr * ones, 0.0, 1.0)]
        + [_disc(n, -1.0, 1.0) for n in (nx, ny, nz)]
    )
    for s in range(NS):
        out_ref[0, s] = slots[s] + (s * ND)


def _discretize_faces(fc4, th2, ph2, fr2):
    grid = (B, G4 // 8)
    return pl.pallas_call(
        _k2_body,
        grid=grid,
        in_specs=[
            pl.BlockSpec((1, 9, 8, 256), lambda b, i: (b, 0, i, 0)),
            pl.BlockSpec(memory_space=pltpu.SMEM),
            pl.BlockSpec(memory_space=pltpu.SMEM),
            pl.BlockSpec(memory_space=pltpu.SMEM),
        ],
        out_specs=pl.BlockSpec((1, NS, 8, 256), lambda b, i: (b, 0, i, 0)),
        out_shape=jax.ShapeDtypeStruct((B, NS, G4, 256), jnp.int32),
    )(fc4, th2, ph2, fr2)


def _k3_body(cidx_hbm, m_hbm, out_hbm,
             cidxb, rows0, rows1, outb0, outb1, sg0, sg1, so0, so1):
    wid = _wid()
    b, j, rel0 = _tile_range(wid)

    def fire(k, rows, sem):
        for s in range(NS):
            pltpu.async_copy(
                m_hbm.at[cidxb.at[s, pl.ds(k * 16, 16)]],
                rows.at[pl.ds(s * 16, 16)], sem)

    def drain_rows(rows, sem):
        pltpu.make_async_copy(m_hbm.at[pl.ds(0, NS * 16)], rows, sem).wait()

    def drain_out(outb, sem):
        pltpu.make_async_copy(outb, out_hbm.at[0, pl.ds(0, 16)], sem).wait()

    def accum(rows, outb):
        def face(f, carry):
            for v in range(8):
                acc = rows[f, pl.ds(v * 16, 16)]
                for s in range(1, NS):
                    acc = acc + rows[16 * s + f, pl.ds(v * 16, 16)]
                outb[f, pl.ds(v * 16, 16)] = acc
            return carry
        lax.fori_loop(0, 16, face, 0)

    def chunk(c, carry):
        off = rel0 + c * 1024
        pltpu.sync_copy(cidx_hbm.at[b, :, pl.ds(off, 1024)], cidxb)
        fire(0, rows0, sg0)
        fire(1, rows1, sg1)

        def pair(i, carry):
            k0 = 2 * i

            @pl.when(i > 0)
            def _():
                drain_out(outb0, so0)
            drain_rows(rows0, sg0)
            accum(rows0, outb0)
            pltpu.async_copy(outb0, out_hbm.at[b, pl.ds(off + k0 * 16, 16)],
                             so0)

            @pl.when(k0 + 2 < 64)
            def _():
                fire(k0 + 2, rows0, sg0)

            @pl.when(i > 0)
            def _():
                drain_out(outb1, so1)
            drain_rows(rows1, sg1)
            accum(rows1, outb1)
            pltpu.async_copy(outb1,
                             out_hbm.at[b, pl.ds(off + (k0 + 1) * 16, 16)],
                             so1)

            @pl.when(k0 + 3 < 64)
            def _():
                fire(k0 + 3, rows1, sg1)
            return carry

        lax.fori_loop(0, 32, pair, 0)
        drain_out(outb0, so0)
        drain_out(outb1, so1)
        return carry

    n_ch = jnp.where(j < 7, 6, 5)
    lax.fori_loop(0, n_ch, chunk, 0)

    off_e = rel0 + n_ch * 1024
    pltpu.sync_copy(cidx_hbm.at[b, :, pl.ds(off_e, 1024)], cidxb)

    def egroup(k, carry):
        fire(k, rows0, sg0)
        drain_rows(rows0, sg0)
        accum(rows0, outb0)
        pltpu.sync_copy(outb0, out_hbm.at[b, pl.ds(off_e + k * 16, 16)])
        return carry

    n_sub = jnp.where(j < 7, 8, _EPI_LAST // 16)
    lax.fori_loop(0, n_sub, egroup, 0)


def _gather_accumulate(cidx, m):
    mesh = plsc.VectorSubcoreMesh(core_axis_name="c", subcore_axis_name="s")
    return pl.kernel(
        _k3_body,
        out_type=jax.ShapeDtypeStruct((B, NF, DIM), jnp.float32),
        mesh=mesh,
        scratch_types=[
            pltpu.VMEM((NS, 1024), jnp.int32),
            pltpu.VMEM((NS * 16, DIM), jnp.float32),
            pltpu.VMEM((NS * 16, DIM), jnp.float32),
            pltpu.VMEM((16, DIM), jnp.float32),
            pltpu.VMEM((16, DIM), jnp.float32),
            pltpu.SemaphoreType.DMA,
            pltpu.SemaphoreType.DMA,
            pltpu.SemaphoreType.DMA,
            pltpu.SemaphoreType.DMA,
        ],
    )(cidx, m)


def kernel(vertices, faces, in_theta, in_phi, in_freq, coor_embed,
           angle_embed, area_embed, emnoangle_embed, emangle_embed,
           emfreq_embed, normal_embed, W, bias):
    tables = jnp.stack(
        [coor_embed] * 9 + [angle_embed] * 3 + [area_embed, emnoangle_embed]
        + [emangle_embed] * 3 + [emfreq_embed] + [normal_embed] * 3, axis=0)
    m = _fold_tables(tables, W.reshape(NS, DE, DIM), bias.reshape(1, DIM))
    faces_flat = jnp.pad(faces, ((0, 0), (0, 176), (0, 0))).reshape(B, -1)
    fc = _gather_face_coords(vertices.reshape(B, NV * 3), faces_flat)
    cidx4 = _discretize_faces(
        fc.reshape(B, 9, G4, 256),
        in_theta.reshape(B, 1), in_phi.reshape(B, 1), in_freq.reshape(B, 1))
    out = _gather_accumulate(cidx4.reshape(B, NS, NF_PAD), m)
    return out


# speedup vs baseline: 15.3171x; 1.3372x over previous
import functools
import math

import jax
import jax.numpy as jnp
from jax import lax
from jax.experimental import pallas as pl
from jax.experimental.pallas import tpu as pltpu
from jax.experimental.pallas import tpu_sc as plsc

B, NV, NF = 4, 25000, 50000
ND, DE, DIM, NS = 128, 16, 128, 21
NF_PAD = 51200
G4 = NF_PAD // 256
PI = float(math.pi)

_N_TILE = 6272
_EPI_LAST = 976


def _wid():
    return lax.axis_index("s") * 2 + lax.axis_index("c")


def _tile_range(wid):
    b = wid // 8
    j = wid % 8
    return b, j, j * _N_TILE


def _k0_body(t_ref, w_ref, bias_ref, m_ref):
    for s in range(NS):
        prod = jnp.dot(t_ref[s], w_ref[s], preferred_element_type=jnp.float32)
        if s == 0:
            prod = prod + bias_ref[0][None, :]
        m_ref[s * ND:(s + 1) * ND, :] = prod


def _fold_tables(tables, w_r, bias2d):
    return pl.pallas_call(
        _k0_body,
        out_shape=jax.ShapeDtypeStruct((NS * ND, DIM), jnp.float32),
    )(tables, w_r, bias2d)


def _k1_body(verts_hbm, faces_hbm, fc_hbm, verts_v, fbuf, cbuf):
    wid = _wid()
    b, j, rel0 = _tile_range(wid)
    pltpu.sync_copy(verts_hbm.at[b], verts_v)

    def gather_group(g, carry):
        i16 = (lax.iota(jnp.int32, 16) + g * 16) * 3
        for p in range(3):
            vid = plsc.load_gather(fbuf, [i16 + p]) * 3
            for c in range(3):
                val = plsc.load_gather(verts_v, [vid + c])
                cbuf[3 * p + c, pl.ds(g * 16, 16)] = val
        return carry

    def chunk(c, carry):
        off = rel0 + c * 1024
        pltpu.sync_copy(faces_hbm.at[b, pl.ds(off * 3, 3072)], fbuf)
        lax.fori_loop(0, 64, gather_group, 0)
        pltpu.sync_copy(cbuf, fc_hbm.at[b, :, pl.ds(off, 1024)])
        return carry

    n_ch = jnp.where(j < 7, 6, 5)
    lax.fori_loop(0, n_ch, chunk, 0)

    off_e = rel0 + n_ch * 1024
    pltpu.sync_copy(faces_hbm.at[b, pl.ds(off_e * 3, 3072)], fbuf)
    n_g = jnp.where(j < 7, 8, _EPI_LAST // 16)
    lax.fori_loop(0, n_g, gather_group, 0)

    @pl.when(j < 7)
    def _():
        pltpu.sync_copy(cbuf.at[:, pl.ds(0, 128)],
                        fc_hbm.at[b, :, pl.ds(off_e, 128)])

    @pl.when(j >= 7)
    def _():
        pltpu.sync_copy(cbuf, fc_hbm.at[b, :, pl.ds(off_e, 1024)])


def _gather_face_coords(vertices, faces):
    mesh = plsc.VectorSubcoreMesh(core_axis_name="c", subcore_axis_name="s")
    return pl.kernel(
        _k1_body,
        out_type=jax.ShapeDtypeStruct((B, 9, NF_PAD), jnp.float32),
        mesh=mesh,
        compiler_params=pltpu.CompilerParams(needs_layout_passes=False),
        scratch_types=[
            pltpu.VMEM((NV * 3,), jnp.float32),
            pltpu.VMEM((3072,), jnp.int32),
            pltpu.VMEM((9, 1024), jnp.float32),
        ],
    )(vertices, faces)


def _disc(t, lo, hi):
    u = (t - lo) / (hi - lo) * 128.0 - 0.5
    r = (u + 12582912.0) - 12582912.0
    r = jnp.clip(r, 0.0, 127.0)
    return jnp.clip(r.astype(jnp.int32), 0, 127)


def _acos(x):
    ax = jnp.abs(x)
    p = -0.0012624911
    for c in (0.0066700901, -0.0170881256, 0.0308918810, -0.0501743046,
              0.0889789874, -0.2145988016, 1.5707963050):
        p = p * ax + c
    r = jnp.sqrt(1.0 - ax) * p
    return jnp.where(x >= 0.0, r, PI - r)


def _k2_body(fc_ref, th_ref, ph_ref, fr_ref, out_ref):
    b = pl.program_id(0)
    fc = fc_ref[0]
    x = [fc[r] for r in range(9)]
    sh = [x[6], x[7], x[8], x[0], x[1], x[2], x[3], x[4], x[5]]
    e = [x[r] - sh[r] for r in range(9)]
    nv = []
    for p in range(3):
        s = e[3 * p] * e[3 * p] + e[3 * p + 1] * e[3 * p + 1] \
            + e[3 * p + 2] * e[3 * p + 2]
        inv = 1.0 / (jnp.sqrt(s) + 1e-12)
        nv += [e[3 * p + c] * inv for c in range(3)]
    ang = []
    for c in range(3):
        c2 = (c + 2) % 3
        nd = -(nv[c] * nv[c2] + nv[3 + c] * nv[3 + c2] + nv[6 + c] * nv[6 + c2])
        nd = jnp.clip(nd, -1.0 + 1e-5, 1.0 - 1e-5)
        ang.append(_acos(nd))
    crx = e[1] * e[5] - e[2] * e[4]
    cry = e[2] * e[3] - e[0] * e[5]
    crz = e[0] * e[4] - e[1] * e[3]
    crn = jnp.sqrt(crx * crx + cry * cry + crz * crz)
    inv = 1.0 / (crn + 1e-12)
    nx, ny, nz = crx * inv, cry * inv, crz * inv
    area = crn * 0.5
    th = th_ref[b, 0] * (PI / 180.0)
    ph = ph_ref[b, 0] * (PI / 180.0)
    shp = jnp.sin(ph)
    ivx, ivy, ivz = shp * jnp.cos(th), shp * jnp.sin(th), jnp.cos(ph)
    ivn = jnp.sqrt(ivx * ivx + ivy * ivy + ivz * ivz) + 1e-12
    jvx, jvy, jvz = ivx / ivn, ivy / ivn, ivz / ivn
    nn = jnp.sqrt(nx * nx + ny * ny + nz * nz) + 1e-12
    nd2 = -((nx / nn) * jvx + (ny / nn) * jvy + (nz / nn) * jvz)
    nd2 = jnp.clip(nd2, -1.0 + 1e-5, 1.0 - 1e-5)
    emno = _acos(nd2)
    fr = fr_ref[b, 0]
    ones = jnp.ones_like(x[0])

    slots = (
        [_disc(x[r], 0.0, 1.0) for r in range(9)]
        + [_disc(a, 0.0, PI) for a in ang]
        + [_disc(area, 0.0, 2.0), _disc(emno, 0.0, PI)]
        + [_disc(iv * ones, -1.0, 1.0) for iv in (ivx, ivy, ivz)]
        + [_disc(fr * ones, 0.0, 1.0)]
        + [_disc(n, -1.0, 1.0) for n in (nx, ny, nz)]
    )
    for s in range(NS):
        out_ref[0, s] = slots[s] + (s * ND)


def _discretize_faces(fc4, th2, ph2, fr2):
    grid = (B, G4 // 8)
    return pl.pallas_call(
        _k2_body,
        grid=grid,
        in_specs=[
            pl.BlockSpec((1, 9, 8, 256), lambda b, i: (b, 0, i, 0)),
            pl.BlockSpec(memory_space=pltpu.SMEM),
            pl.BlockSpec(memory_space=pltpu.SMEM),
            pl.BlockSpec(memory_space=pltpu.SMEM),
        ],
        out_specs=pl.BlockSpec((1, NS, 8, 256), lambda b, i: (b, 0, i, 0)),
        out_shape=jax.ShapeDtypeStruct((B, NS, G4, 256), jnp.int32),
    )(fc4, th2, ph2, fr2)


def _split(w):
    wi = plsc.bitcast(w, jnp.int32)
    lo = plsc.bitcast(wi << 16, jnp.float32)
    hi = plsc.bitcast(wi & jnp.int32(-65536), jnp.float32)
    return lo, hi


def _k3_body(cidx_hbm, mp_hbm, out_hbm,
             cidxb, rows0, rows1, outb0, outb1, sg0, sg1, so0, so1):
    wid = _wid()
    b, j, rel0 = _tile_range(wid)

    def fire(k, nf, rows, sem):
        for s in range(NS):
            pltpu.async_copy(
                mp_hbm.at[cidxb.at[s, pl.ds(k * 32, nf)]],
                rows.at[pl.ds(s * nf, nf)], sem)

    def drain_rows(nrows, rows, sem):
        pltpu.make_async_copy(mp_hbm.at[pl.ds(0, nrows)],
                              rows.at[pl.ds(0, nrows)], sem).wait()

    def drain_out(nf, outb, sem):
        pltpu.make_async_copy(outb.at[pl.ds(0, nf)],
                              out_hbm.at[0, pl.ds(0, nf)], sem).wait()

    def make_accum(nf):
        def accum(rows, outb):
            def face(f, carry):
                for v in range(4):
                    lo, hi = _split(rows[f, pl.ds(v * 16, 16)])
                    for s in range(1, NS):
                        l2, h2 = _split(rows[nf * s + f, pl.ds(v * 16, 16)])
                        lo = lo + l2
                        hi = hi + h2
                    outb[f, pl.ds(v * 16, 16)] = lo
                    outb[f, pl.ds(64 + v * 16, 16)] = hi
                return carry
            lax.fori_loop(0, nf, face, 0)
        return accum

    accum32 = make_accum(32)
    accum16 = make_accum(16)

    def chunk(c, carry):
        off = rel0 + c * 1024
        pltpu.sync_copy(cidx_hbm.at[b, :, pl.ds(off, 1024)], cidxb)
        fire(0, 32, rows0, sg0)
        fire(1, 32, rows1, sg1)

        def pair(i, carry):
            k0 = 2 * i

            @pl.when(i > 0)
            def _():
                drain_out(32, outb0, so0)
            drain_rows(NS * 32, rows0, sg0)
            accum32(rows0, outb0)
            pltpu.async_copy(outb0, out_hbm.at[b, pl.ds(off + k0 * 32, 32)],
                             so0)

            @pl.when(k0 + 2 < 32)
            def _():
                fire(k0 + 2, 32, rows0, sg0)

            @pl.when(i > 0)
            def _():
                drain_out(32, outb1, so1)
            drain_rows(NS * 32, rows1, sg1)
            accum32(rows1, outb1)
            pltpu.async_copy(outb1,
                             out_hbm.at[b, pl.ds(off + (k0 + 1) * 32, 32)],
                             so1)

            @pl.when(k0 + 3 < 32)
            def _():
                fire(k0 + 3, 32, rows1, sg1)
            return carry

        lax.fori_loop(0, 16, pair, 0)
        drain_out(32, outb0, so0)
        drain_out(32, outb1, so1)
        return carry

    n_ch = jnp.where(j < 7, 6, 5)
    lax.fori_loop(0, n_ch, chunk, 0)

    off_e = rel0 + n_ch * 1024
    pltpu.sync_copy(cidx_hbm.at[b, :, pl.ds(off_e, 1024)], cidxb)

    def egroup(k, carry):
        fire(k, 32, rows0, sg0)
        drain_rows(NS * 32, rows0, sg0)
        accum32(rows0, outb0)
        pltpu.sync_copy(outb0, out_hbm.at[b, pl.ds(off_e + k * 32, 32)])
        return carry

    n_sub = jnp.where(j < 7, 4, 30)
    lax.fori_loop(0, n_sub, egroup, 0)

    @pl.when(j >= 7)
    def _():
        for s in range(NS):
            pltpu.async_copy(
                mp_hbm.at[cidxb.at[s, pl.ds(960, 16)]],
                rows0.at[pl.ds(s * 16, 16)], sg0)
        drain_rows(NS * 16, rows0, sg0)
        accum16(rows0, outb0)
        pltpu.sync_copy(outb0.at[pl.ds(0, 16)],
                        out_hbm.at[b, pl.ds(off_e + 960, 16)])


def _gather_accumulate(cidx, mp):
    mesh = plsc.VectorSubcoreMesh(core_axis_name="c", subcore_axis_name="s")
    return pl.kernel(
        _k3_body,
        out_type=jax.ShapeDtypeStruct((B, NF, DIM), jnp.float32),
        mesh=mesh,
        compiler_params=pltpu.CompilerParams(needs_layout_passes=False,
                                             use_tc_tiling_on_sc=False),
        scratch_types=[
            pltpu.VMEM((NS, 1024), jnp.int32),
            pltpu.VMEM((NS * 32, 64), jnp.float32),
            pltpu.VMEM((NS * 32, 64), jnp.float32),
            pltpu.VMEM((32, DIM), jnp.float32),
            pltpu.VMEM((32, DIM), jnp.float32),
            pltpu.SemaphoreType.DMA,
            pltpu.SemaphoreType.DMA,
            pltpu.SemaphoreType.DMA,
            pltpu.SemaphoreType.DMA,
        ],
    )(cidx, mp)


def kernel(vertices, faces, in_theta, in_phi, in_freq, coor_embed,
           angle_embed, area_embed, emnoangle_embed, emangle_embed,
           emfreq_embed, normal_embed, W, bias):
    tables = jnp.stack(
        [coor_embed] * 9 + [angle_embed] * 3 + [area_embed, emnoangle_embed]
        + [emangle_embed] * 3 + [emfreq_embed] + [normal_embed] * 3, axis=0)
    m = _fold_tables(tables, W.reshape(NS, DE, DIM), bias.reshape(1, DIM))
    faces_flat = jnp.pad(faces, ((0, 0), (0, 176), (0, 0))).reshape(B, -1)
    fc = _gather_face_coords(vertices.reshape(B, NV * 3), faces_flat)
    cidx4 = _discretize_faces(
        fc.reshape(B, 9, G4, 256),
        in_theta.reshape(B, 1), in_phi.reshape(B, 1), in_freq.reshape(B, 1))
    mb = m.astype(jnp.bfloat16)
    mp = lax.bitcast_convert_type(
        jnp.stack([mb[:, :64], mb[:, 64:]], axis=-1), jnp.float32)
    out = _gather_accumulate(cidx4.reshape(B, NS, NF_PAD), mp)
    return out


# speedup vs baseline: 23.8938x; 1.5599x over previous
import functools
import math

import jax
import jax.numpy as jnp
from jax import lax
from jax.experimental import pallas as pl
from jax.experimental.pallas import tpu as pltpu
from jax.experimental.pallas import tpu_sc as plsc

B, NV, NF = 4, 25000, 50000
ND, DE, DIM, NS = 128, 16, 128, 21
NF_PAD = 51200
G4 = NF_PAD // 256
PI = float(math.pi)

_N_TILE = 6272
_EPI_LAST = 976


def _wid():
    return lax.axis_index("s") * 2 + lax.axis_index("c")


def _tile_range(wid):
    b = wid // 8
    j = wid % 8
    return b, j, j * _N_TILE


def _disc_scalar(t, lo, hi):
    u = (t - lo) / (hi - lo) * 128.0 - 0.5
    r = (u + 12582912.0) - 12582912.0
    r = jnp.clip(r, 0.0, 127.0)
    return jnp.clip(r.astype(jnp.int32), 0, 127)


def _k0_body(t_ref, w_ref, bias_ref, th_ref, ph_ref, fr_ref, m_ref, r_ref):
    for s in range(NS):
        prod = jnp.dot(t_ref[s], w_ref[s], preferred_element_type=jnp.float32)
        if s == 0:
            prod = prod + bias_ref[0][None, :]
        m_ref[s * ND:(s + 1) * ND, :] = prod
    for bb in range(B):
        th = th_ref[bb, 0] * (PI / 180.0)
        ph = ph_ref[bb, 0] * (PI / 180.0)
        shp = jnp.sin(ph)
        iv = (shp * jnp.cos(th), shp * jnp.sin(th), jnp.cos(ph))
        ds = [_disc_scalar(v, -1.0, 1.0) for v in iv]
        ds.append(_disc_scalar(fr_ref[bb, 0], 0.0, 1.0))
        acc = None
        iota = lax.broadcasted_iota(jnp.int32, (1, ND), 1)
        for k, s in enumerate((14, 15, 16, 17)):
            oh = (iota == ds[k]).astype(jnp.float32)
            row = jnp.dot(oh, t_ref[s], preferred_element_type=jnp.float32)
            contrib = jnp.dot(row, w_ref[s],
                              preferred_element_type=jnp.float32)
            acc = contrib if acc is None else acc + contrib
        r_ref[pl.ds(bb, 1), :] = acc


def _fold_tables(tables, w_r, bias2d, th2, ph2, fr2):
    return pl.pallas_call(
        _k0_body,
        in_specs=[
            pl.BlockSpec(),
            pl.BlockSpec(),
            pl.BlockSpec(),
            pl.BlockSpec(memory_space=pltpu.SMEM),
            pl.BlockSpec(memory_space=pltpu.SMEM),
            pl.BlockSpec(memory_space=pltpu.SMEM),
        ],
        out_shape=[jax.ShapeDtypeStruct((NS * ND, DIM), jnp.float32),
                   jax.ShapeDtypeStruct((B, DIM), jnp.float32)],
    )(tables, w_r, bias2d, th2, ph2, fr2)


def _k1_body(verts_hbm, faces_hbm, fc_hbm, verts_v, fbuf, cbuf):
    wid = _wid()
    b, j, rel0 = _tile_range(wid)
    pltpu.sync_copy(verts_hbm.at[b], verts_v)

    def gather_group(g, carry):
        i16 = (lax.iota(jnp.int32, 16) + g * 16) * 3
        for p in range(3):
            vid = plsc.load_gather(fbuf, [i16 + p]) * 3
            for c in range(3):
                val = plsc.load_gather(verts_v, [vid + c])
                cbuf[3 * p + c, pl.ds(g * 16, 16)] = val
        return carry

    def chunk(c, carry):
        off = rel0 + c * 1024
        pltpu.sync_copy(faces_hbm.at[b, pl.ds(off * 3, 3072)], fbuf)
        lax.fori_loop(0, 64, gather_group, 0)
        pltpu.sync_copy(cbuf, fc_hbm.at[b, :, pl.ds(off, 1024)])
        return carry

    n_ch = jnp.where(j < 7, 6, 5)
    lax.fori_loop(0, n_ch, chunk, 0)

    off_e = rel0 + n_ch * 1024
    pltpu.sync_copy(faces_hbm.at[b, pl.ds(off_e * 3, 3072)], fbuf)
    n_g = jnp.where(j < 7, 8, _EPI_LAST // 16)
    lax.fori_loop(0, n_g, gather_group, 0)

    @pl.when(j < 7)
    def _():
        pltpu.sync_copy(cbuf.at[:, pl.ds(0, 128)],
                        fc_hbm.at[b, :, pl.ds(off_e, 128)])

    @pl.when(j >= 7)
    def _():
        pltpu.sync_copy(cbuf, fc_hbm.at[b, :, pl.ds(off_e, 1024)])


def _gather_face_coords(vertices, faces):
    mesh = plsc.VectorSubcoreMesh(core_axis_name="c", subcore_axis_name="s")
    return pl.kernel(
        _k1_body,
        out_type=jax.ShapeDtypeStruct((B, 9, NF_PAD), jnp.float32),
        mesh=mesh,
        compiler_params=pltpu.CompilerParams(needs_layout_passes=False),
        scratch_types=[
            pltpu.VMEM((NV * 3,), jnp.float32),
            pltpu.VMEM((3072,), jnp.int32),
            pltpu.VMEM((9, 1024), jnp.float32),
        ],
    )(vertices, faces)


def _disc(t, lo, hi):
    u = (t - lo) / (hi - lo) * 128.0 - 0.5
    r = (u + 12582912.0) - 12582912.0
    r = jnp.clip(r, 0.0, 127.0)
    return jnp.clip(r.astype(jnp.int32), 0, 127)


def _acos(x):
    ax = jnp.abs(x)
    p = -0.0012624911
    for c in (0.0066700901, -0.0170881256, 0.0308918810, -0.0501743046,
              0.0889789874, -0.2145988016, 1.5707963050):
        p = p * ax + c
    r = jnp.sqrt(1.0 - ax) * p
    return jnp.where(x >= 0.0, r, PI - r)


def _k2_body(fc_ref, th_ref, ph_ref, fr_ref, out_ref):
    b = pl.program_id(0)
    fc = fc_ref[0]
    x = [fc[r] for r in range(9)]
    sh = [x[6], x[7], x[8], x[0], x[1], x[2], x[3], x[4], x[5]]
    e = [x[r] - sh[r] for r in range(9)]
    nv = []
    for p in range(3):
        s = e[3 * p] * e[3 * p] + e[3 * p + 1] * e[3 * p + 1] \
            + e[3 * p + 2] * e[3 * p + 2]
        inv = 1.0 / (jnp.sqrt(s) + 1e-12)
        nv += [e[3 * p + c] * inv for c in range(3)]
    ang = []
    for c in range(3):
        c2 = (c + 2) % 3
        nd = -(nv[c] * nv[c2] + nv[3 + c] * nv[3 + c2] + nv[6 + c] * nv[6 + c2])
        nd = jnp.clip(nd, -1.0 + 1e-5, 1.0 - 1e-5)
        ang.append(_acos(nd))
    crx = e[1] * e[5] - e[2] * e[4]
    cry = e[2] * e[3] - e[0] * e[5]
    crz = e[0] * e[4] - e[1] * e[3]
    crn = jnp.sqrt(crx * crx + cry * cry + crz * crz)
    inv = 1.0 / (crn + 1e-12)
    nx, ny, nz = crx * inv, cry * inv, crz * inv
    area = crn * 0.5
    th = th_ref[b, 0] * (PI / 180.0)
    ph = ph_ref[b, 0] * (PI / 180.0)
    shp = jnp.sin(ph)
    ivx, ivy, ivz = shp * jnp.cos(th), shp * jnp.sin(th), jnp.cos(ph)
    ivn = jnp.sqrt(ivx * ivx + ivy * ivy + ivz * ivz) + 1e-12
    jvx, jvy, jvz = ivx / ivn, ivy / ivn, ivz / ivn
    nn = jnp.sqrt(nx * nx + ny * ny + nz * nz) + 1e-12
    nd2 = -((nx / nn) * jvx + (ny / nn) * jvy + (nz / nn) * jvz)
    nd2 = jnp.clip(nd2, -1.0 + 1e-5, 1.0 - 1e-5)
    emno = _acos(nd2)
    fr = fr_ref[b, 0]
    ones = jnp.ones_like(x[0])

    slots = (
        [_disc(x[r], 0.0, 1.0) for r in range(9)]
        + [_disc(a, 0.0, PI) for a in ang]
        + [_disc(area, 0.0, 2.0), _disc(emno, 0.0, PI)]
        + [_disc(iv * ones, -1.0, 1.0) for iv in (ivx, ivy, ivz)]
        + [_disc(fr * ones, 0.0, 1.0)]
        + [_disc(n, -1.0, 1.0) for n in (nx, ny, nz)]
    )
    for s in range(NS):
        out_ref[0, s] = slots[s] + (s * ND)


def _discretize_faces(fc4, th2, ph2, fr2):
    grid = (B, G4 // 8)
    return pl.pallas_call(
        _k2_body,
        grid=grid,
        in_specs=[
            pl.BlockSpec((1, 9, 8, 256), lambda b, i: (b, 0, i, 0)),
            pl.BlockSpec(memory_space=pltpu.SMEM),
            pl.BlockSpec(memory_space=pltpu.SMEM),
            pl.BlockSpec(memory_space=pltpu.SMEM),
        ],
        out_specs=pl.BlockSpec((1, NS, 8, 256), lambda b, i: (b, 0, i, 0)),
        out_shape=jax.ShapeDtypeStruct((B, NS, G4, 256), jnp.int32),
    )(fc4, th2, ph2, fr2)


def _split(w):
    wi = plsc.bitcast(w, jnp.int32)
    lo = plsc.bitcast(wi << 16, jnp.float32)
    hi = plsc.bitcast(wi & jnp.int32(-65536), jnp.float32)
    return lo, hi


_SLOTS = tuple(range(14)) + (18, 19, 20)
_NSL = len(_SLOTS)


def _k3_body(cidx_hbm, mp_hbm, r_hbm, out_hbm,
             cidxb, rows0, rows1, outb0, outb1, rbuf, sg0, sg1, so0, so1):
    wid = _wid()
    b, j, rel0 = _tile_range(wid)
    pltpu.sync_copy(r_hbm.at[b], rbuf)

    def fire(k, nf, rows, sem):
        for si, s in enumerate(_SLOTS):
            pltpu.async_copy(
                mp_hbm.at[cidxb.at[s, pl.ds(k * 32, nf)]],
                rows.at[pl.ds(si * nf, nf)], sem)

    def drain_rows(nrows, rows, sem):
        pltpu.make_async_copy(mp_hbm.at[pl.ds(0, nrows)],
                              rows.at[pl.ds(0, nrows)], sem).wait()

    def drain_out(nf, outb, sem):
        pltpu.make_async_copy(outb.at[pl.ds(0, nf)],
                              out_hbm.at[0, pl.ds(0, nf)], sem).wait()

    def make_accum(nf):
        def accum(rows, outb):
            def face(f, carry):
                for v in range(4):
                    lo = rbuf[pl.ds(v * 16, 16)]
                    hi = rbuf[pl.ds(64 + v * 16, 16)]
                    for si in range(_NSL):
                        l2, h2 = _split(rows[nf * si + f, pl.ds(v * 16, 16)])
                        lo = lo + l2
                        hi = hi + h2
                    outb[f, pl.ds(v * 16, 16)] = lo
                    outb[f, pl.ds(64 + v * 16, 16)] = hi
                return carry
            lax.fori_loop(0, nf, face, 0)
        return accum

    accum32 = make_accum(32)
    accum16 = make_accum(16)

    def chunk(c, carry):
        off = rel0 + c * 1024
        pltpu.sync_copy(cidx_hbm.at[b, :, pl.ds(off, 1024)], cidxb)
        fire(0, 32, rows0, sg0)
        fire(1, 32, rows1, sg1)

        def pair(i, carry):
            k0 = 2 * i

            @pl.when(i > 0)
            def _():
                drain_out(32, outb0, so0)
            drain_rows(_NSL * 32, rows0, sg0)
            accum32(rows0, outb0)
            pltpu.async_copy(outb0, out_hbm.at[b, pl.ds(off + k0 * 32, 32)],
                             so0)

            @pl.when(k0 + 2 < 32)
            def _():
                fire(k0 + 2, 32, rows0, sg0)

            @pl.when(i > 0)
            def _():
                drain_out(32, outb1, so1)
            drain_rows(_NSL * 32, rows1, sg1)
            accum32(rows1, outb1)
            pltpu.async_copy(outb1,
                             out_hbm.at[b, pl.ds(off + (k0 + 1) * 32, 32)],
                             so1)

            @pl.when(k0 + 3 < 32)
            def _():
                fire(k0 + 3, 32, rows1, sg1)
            return carry

        lax.fori_loop(0, 16, pair, 0)
        drain_out(32, outb0, so0)
        drain_out(32, outb1, so1)
        return carry

    n_ch = jnp.where(j < 7, 6, 5)
    lax.fori_loop(0, n_ch, chunk, 0)

    off_e = rel0 + n_ch * 1024
    pltpu.sync_copy(cidx_hbm.at[b, :, pl.ds(off_e, 1024)], cidxb)

    def egroup(k, carry):
        fire(k, 32, rows0, sg0)
        drain_rows(_NSL * 32, rows0, sg0)
        accum32(rows0, outb0)
        pltpu.sync_copy(outb0, out_hbm.at[b, pl.ds(off_e + k * 32, 32)])
        return carry

    n_sub = jnp.where(j < 7, 4, 30)
    lax.fori_loop(0, n_sub, egroup, 0)

    @pl.when(j >= 7)
    def _():
        for si, s in enumerate(_SLOTS):
            pltpu.async_copy(
                mp_hbm.at[cidxb.at[s, pl.ds(960, 16)]],
                rows0.at[pl.ds(si * 16, 16)], sg0)
        drain_rows(_NSL * 16, rows0, sg0)
        accum16(rows0, outb0)
        pltpu.sync_copy(outb0.at[pl.ds(0, 16)],
                        out_hbm.at[b, pl.ds(off_e + 960, 16)])


def _gather_accumulate(cidx, mp, r):
    mesh = plsc.VectorSubcoreMesh(core_axis_name="c", subcore_axis_name="s")
    return pl.kernel(
        _k3_body,
        out_type=jax.ShapeDtypeStruct((B, NF, DIM), jnp.float32),
        mesh=mesh,
        compiler_params=pltpu.CompilerParams(needs_layout_passes=False,
                                             use_tc_tiling_on_sc=False),
        scratch_types=[
            pltpu.VMEM((NS, 1024), jnp.int32),
            pltpu.VMEM((_NSL * 32, 64), jnp.float32),
            pltpu.VMEM((_NSL * 32, 64), jnp.float32),
            pltpu.VMEM((32, DIM), jnp.float32),
            pltpu.VMEM((32, DIM), jnp.float32),
            pltpu.VMEM((DIM,), jnp.float32),
            pltpu.SemaphoreType.DMA,
            pltpu.SemaphoreType.DMA,
            pltpu.SemaphoreType.DMA,
            pltpu.SemaphoreType.DMA,
        ],
    )(cidx, mp, r)


def kernel(vertices, faces, in_theta, in_phi, in_freq, coor_embed,
           angle_embed, area_embed, emnoangle_embed, emangle_embed,
           emfreq_embed, normal_embed, W, bias):
    tables = jnp.stack(
        [coor_embed] * 9 + [angle_embed] * 3 + [area_embed, emnoangle_embed]
        + [emangle_embed] * 3 + [emfreq_embed] + [normal_embed] * 3, axis=0)
    th2 = in_theta.reshape(B, 1)
    ph2 = in_phi.reshape(B, 1)
    fr2 = in_freq.reshape(B, 1)
    m, r = _fold_tables(tables, W.reshape(NS, DE, DIM), bias.reshape(1, DIM),
                        th2, ph2, fr2)
    faces_flat = jnp.pad(faces, ((0, 0), (0, 176), (0, 0))).reshape(B, -1)
    fc = _gather_face_coords(vertices.reshape(B, NV * 3), faces_flat)
    cidx4 = _discretize_faces(fc.reshape(B, 9, G4, 256), th2, ph2, fr2)
    mb = m.astype(jnp.bfloat16)
    mp = lax.bitcast_convert_type(
        jnp.stack([mb[:, :64], mb[:, 64:]], axis=-1), jnp.float32)
    out = _gather_accumulate(cidx4.reshape(B, NS, NF_PAD), mp, r)
    return out


# speedup vs baseline: 24.1738x; 1.0117x over previous
import functools
import math

import jax
import jax.numpy as jnp
from jax import lax
from jax.experimental import pallas as pl
from jax.experimental.pallas import tpu as pltpu
from jax.experimental.pallas import tpu_sc as plsc

B, NV, NF = 4, 25000, 50000
ND, DE, DIM, NS = 128, 16, 128, 21
NF_PAD = 51200
G4 = NF_PAD // 256
PI = float(math.pi)

_N_TILE = 6272
_EPI_LAST = 976


def _wid():
    return lax.axis_index("s") * 2 + lax.axis_index("c")


def _tile_range(wid):
    b = wid // 8
    j = wid % 8
    return b, j, j * _N_TILE


def _disc_scalar(t, lo, hi):
    u = (t - lo) / (hi - lo) * 128.0 - 0.5
    r = (u + 12582912.0) - 12582912.0
    r = jnp.clip(r, 0.0, 127.0)
    return jnp.clip(r.astype(jnp.int32), 0, 127)


def _k0_body(t_ref, w_ref, bias_ref, th_ref, ph_ref, fr_ref, m_ref, r_ref):
    for s in range(NS):
        prod = jnp.dot(t_ref[s], w_ref[s], preferred_element_type=jnp.float32)
        if s == 0:
            prod = prod + bias_ref[0][None, :]
        m_ref[s * ND:(s + 1) * ND, :] = prod
    for bb in range(B):
        th = th_ref[bb, 0] * (PI / 180.0)
        ph = ph_ref[bb, 0] * (PI / 180.0)
        shp = jnp.sin(ph)
        iv = (shp * jnp.cos(th), shp * jnp.sin(th), jnp.cos(ph))
        ds = [_disc_scalar(v, -1.0, 1.0) for v in iv]
        ds.append(_disc_scalar(fr_ref[bb, 0], 0.0, 1.0))
        acc = None
        iota = lax.broadcasted_iota(jnp.int32, (1, ND), 1)
        for k, s in enumerate((14, 15, 16, 17)):
            oh = (iota == ds[k]).astype(jnp.float32)
            row = jnp.dot(oh, t_ref[s], preferred_element_type=jnp.float32)
            contrib = jnp.dot(row, w_ref[s],
                              preferred_element_type=jnp.float32)
            acc = contrib if acc is None else acc + contrib
        r_ref[pl.ds(bb, 1), :] = acc


def _fold_tables(tables, w_r, bias2d, th2, ph2, fr2):
    return pl.pallas_call(
        _k0_body,
        in_specs=[
            pl.BlockSpec(),
            pl.BlockSpec(),
            pl.BlockSpec(),
            pl.BlockSpec(memory_space=pltpu.SMEM),
            pl.BlockSpec(memory_space=pltpu.SMEM),
            pl.BlockSpec(memory_space=pltpu.SMEM),
        ],
        out_shape=[jax.ShapeDtypeStruct((NS * ND, DIM), jnp.float32),
                   jax.ShapeDtypeStruct((B, DIM), jnp.float32)],
    )(tables, w_r, bias2d, th2, ph2, fr2)


def _k1_body(verts_hbm, faces_hbm, fc_hbm, verts_v, fbuf, cbuf):
    wid = _wid()
    b, j, rel0 = _tile_range(wid)
    pltpu.sync_copy(verts_hbm.at[b], verts_v)

    def gather_group(g, carry):
        i16 = (lax.iota(jnp.int32, 16) + g * 16) * 3
        for p in range(3):
            vid = plsc.load_gather(fbuf, [i16 + p]) * 3
            for c in range(3):
                val = plsc.load_gather(verts_v, [vid + c])
                cbuf[3 * p + c, pl.ds(g * 16, 16)] = val
        return carry

    def chunk(c, carry):
        off = rel0 + c * 1024
        pltpu.sync_copy(faces_hbm.at[b, pl.ds(off * 3, 3072)], fbuf)
        lax.fori_loop(0, 64, gather_group, 0)
        pltpu.sync_copy(cbuf, fc_hbm.at[b, :, pl.ds(off, 1024)])
        return carry

    n_ch = jnp.where(j < 7, 6, 5)
    lax.fori_loop(0, n_ch, chunk, 0)

    off_e = rel0 + n_ch * 1024
    pltpu.sync_copy(faces_hbm.at[b, pl.ds(off_e * 3, 3072)], fbuf)
    n_g = jnp.where(j < 7, 8, _EPI_LAST // 16)
    lax.fori_loop(0, n_g, gather_group, 0)

    @pl.when(j < 7)
    def _():
        pltpu.sync_copy(cbuf.at[:, pl.ds(0, 128)],
                        fc_hbm.at[b, :, pl.ds(off_e, 128)])

    @pl.when(j >= 7)
    def _():
        pltpu.sync_copy(cbuf, fc_hbm.at[b, :, pl.ds(off_e, 1024)])


def _gather_face_coords(vertices, faces):
    mesh = plsc.VectorSubcoreMesh(core_axis_name="c", subcore_axis_name="s")
    return pl.kernel(
        _k1_body,
        out_type=jax.ShapeDtypeStruct((B, 9, NF_PAD), jnp.float32),
        mesh=mesh,
        compiler_params=pltpu.CompilerParams(needs_layout_passes=False),
        scratch_types=[
            pltpu.VMEM((NV * 3,), jnp.float32),
            pltpu.VMEM((3072,), jnp.int32),
            pltpu.VMEM((9, 1024), jnp.float32),
        ],
    )(vertices, faces)


def _disc(t, lo, hi):
    u = (t - lo) / (hi - lo) * 128.0 - 0.5
    r = (u + 12582912.0) - 12582912.0
    r = jnp.clip(r, 0.0, 127.0)
    return jnp.clip(r.astype(jnp.int32), 0, 127)


def _acos(x):
    ax = jnp.abs(x)
    p = -0.0012624911
    for c in (0.0066700901, -0.0170881256, 0.0308918810, -0.0501743046,
              0.0889789874, -0.2145988016, 1.5707963050):
        p = p * ax + c
    r = jnp.sqrt(1.0 - ax) * p
    return jnp.where(x >= 0.0, r, PI - r)


def _k2_body(fc_ref, th_ref, ph_ref, fr_ref, out_ref):
    b = pl.program_id(0)
    fc = fc_ref[0]
    x = [fc[r] for r in range(9)]
    sh = [x[6], x[7], x[8], x[0], x[1], x[2], x[3], x[4], x[5]]
    e = [x[r] - sh[r] for r in range(9)]
    nv = []
    for p in range(3):
        s = e[3 * p] * e[3 * p] + e[3 * p + 1] * e[3 * p + 1] \
            + e[3 * p + 2] * e[3 * p + 2]
        inv = 1.0 / (jnp.sqrt(s) + 1e-12)
        nv += [e[3 * p + c] * inv for c in range(3)]
    ang = []
    for c in range(3):
        c2 = (c + 2) % 3
        nd = -(nv[c] * nv[c2] + nv[3 + c] * nv[3 + c2] + nv[6 + c] * nv[6 + c2])
        nd = jnp.clip(nd, -1.0 + 1e-5, 1.0 - 1e-5)
        ang.append(_acos(nd))
    crx = e[1] * e[5] - e[2] * e[4]
    cry = e[2] * e[3] - e[0] * e[5]
    crz = e[0] * e[4] - e[1] * e[3]
    crn = jnp.sqrt(crx * crx + cry * cry + crz * crz)
    inv = 1.0 / (crn + 1e-12)
    nx, ny, nz = crx * inv, cry * inv, crz * inv
    area = crn * 0.5
    th = th_ref[b, 0] * (PI / 180.0)
    ph = ph_ref[b, 0] * (PI / 180.0)
    shp = jnp.sin(ph)
    ivx, ivy, ivz = shp * jnp.cos(th), shp * jnp.sin(th), jnp.cos(ph)
    ivn = jnp.sqrt(ivx * ivx + ivy * ivy + ivz * ivz) + 1e-12
    jvx, jvy, jvz = ivx / ivn, ivy / ivn, ivz / ivn
    nn = jnp.sqrt(nx * nx + ny * ny + nz * nz) + 1e-12
    nd2 = -((nx / nn) * jvx + (ny / nn) * jvy + (nz / nn) * jvz)
    nd2 = jnp.clip(nd2, -1.0 + 1e-5, 1.0 - 1e-5)
    emno = _acos(nd2)
    fr = fr_ref[b, 0]
    ones = jnp.ones_like(x[0])

    slots = (
        [_disc(x[r], 0.0, 1.0) for r in range(9)]
        + [_disc(a, 0.0, PI) for a in ang]
        + [_disc(area, 0.0, 2.0), _disc(emno, 0.0, PI)]
        + [_disc(iv * ones, -1.0, 1.0) for iv in (ivx, ivy, ivz)]
        + [_disc(fr * ones, 0.0, 1.0)]
        + [_disc(n, -1.0, 1.0) for n in (nx, ny, nz)]
    )
    for s in range(NS):
        out_ref[0, s] = slots[s] + (s * ND)


def _discretize_faces(fc4, th2, ph2, fr2):
    grid = (B, G4 // 8)
    return pl.pallas_call(
        _k2_body,
        grid=grid,
        in_specs=[
            pl.BlockSpec((1, 9, 8, 256), lambda b, i: (b, 0, i, 0)),
            pl.BlockSpec(memory_space=pltpu.SMEM),
            pl.BlockSpec(memory_space=pltpu.SMEM),
            pl.BlockSpec(memory_space=pltpu.SMEM),
        ],
        out_specs=pl.BlockSpec((1, NS, 8, 256), lambda b, i: (b, 0, i, 0)),
        out_shape=jax.ShapeDtypeStruct((B, NS, G4, 256), jnp.int32),
    )(fc4, th2, ph2, fr2)


def _split(w):
    wi = plsc.bitcast(w, jnp.int32)
    lo = plsc.bitcast(wi << 16, jnp.float32)
    hi = plsc.bitcast(wi & jnp.int32(-65536), jnp.float32)
    return lo, hi


_SLOTS = tuple(range(14)) + (18, 19, 20)
_NSL = len(_SLOTS)


def _k3_body(cidx_hbm, mp_hbm, r_hbm, out_hbm,
             cidxb, rows0, rows1, rows2, outb0, outb1, outb2, rbuf,
             sg0, sg1, sg2, so0, so1, so2):
    wid = _wid()
    b, j, rel0 = _tile_range(wid)
    pltpu.sync_copy(r_hbm.at[b], rbuf)
    rows_b = (rows0, rows1, rows2)
    outs_b = (outb0, outb1, outb2)
    sg_b = (sg0, sg1, sg2)
    so_b = (so0, so1, so2)

    def fire(k, nf, rows, sem):
        for si, s in enumerate(_SLOTS):
            pltpu.async_copy(
                mp_hbm.at[cidxb.at[s, pl.ds(k * 32, nf)]],
                rows.at[pl.ds(si * nf, nf)], sem)

    def drain_rows(nrows, rows, sem):
        pltpu.make_async_copy(mp_hbm.at[pl.ds(0, nrows)],
                              rows.at[pl.ds(0, nrows)], sem).wait()

    def drain_out(nf, outb, sem):
        pltpu.make_async_copy(outb.at[pl.ds(0, nf)],
                              out_hbm.at[0, pl.ds(0, nf)], sem).wait()

    def make_accum(nf):
        def accum(rows, outb):
            def face(f, carry):
                for v in range(4):
                    lo = rbuf[pl.ds(v * 16, 16)]
                    hi = rbuf[pl.ds(64 + v * 16, 16)]
                    for si in range(_NSL):
                        l2, h2 = _split(rows[nf * si + f, pl.ds(v * 16, 16)])
                        lo = lo + l2
                        hi = hi + h2
                    outb[f, pl.ds(v * 16, 16)] = lo
                    outb[f, pl.ds(64 + v * 16, 16)] = hi
                return carry
            lax.fori_loop(0, nf, face, 0)
        return accum

    accum32 = make_accum(32)
    accum16 = make_accum(16)

    def chunk(c, carry):
        off = rel0 + c * 512
        pltpu.sync_copy(cidx_hbm.at[b, :, pl.ds(off, 512)], cidxb)
        for q in range(3):
            fire(q, 32, rows_b[q], sg_b[q])

        def triple(i, carry):
            k0 = 3 * i
            for q in range(3):
                k = k0 + q

                @pl.when(i > 0)
                def _(q=q):
                    drain_out(32, outs_b[q], so_b[q])
                drain_rows(_NSL * 32, rows_b[q], sg_b[q])
                accum32(rows_b[q], outs_b[q])
                pltpu.async_copy(outs_b[q],
                                 out_hbm.at[b, pl.ds(off + k * 32, 32)],
                                 so_b[q])

                @pl.when(k + 3 < 16)
                def _(q=q, k=k):
                    fire(k + 3, 32, rows_b[q], sg_b[q])
            return carry

        lax.fori_loop(0, 5, triple, 0)
        drain_out(32, outb0, so0)
        drain_rows(_NSL * 32, rows0, sg0)
        accum32(rows0, outb0)
        pltpu.async_copy(outb0, out_hbm.at[b, pl.ds(off + 15 * 32, 32)], so0)
        drain_out(32, outb0, so0)
        drain_out(32, outb1, so1)
        drain_out(32, outb2, so2)
        return carry

    n_ch = jnp.where(j < 7, 12, 11)
    lax.fori_loop(0, n_ch, chunk, 0)

    off_e = rel0 + n_ch * 512
    pltpu.sync_copy(cidx_hbm.at[b, :, pl.ds(off_e, 512)], cidxb)

    def egroup(k, carry):
        fire(k, 32, rows0, sg0)
        drain_rows(_NSL * 32, rows0, sg0)
        accum32(rows0, outb0)
        pltpu.sync_copy(outb0, out_hbm.at[b, pl.ds(off_e + k * 32, 32)])
        return carry

    n_sub = jnp.where(j < 7, 4, 14)
    lax.fori_loop(0, n_sub, egroup, 0)

    @pl.when(j >= 7)
    def _():
        for si, s in enumerate(_SLOTS):
            pltpu.async_copy(
                mp_hbm.at[cidxb.at[s, pl.ds(448, 16)]],
                rows0.at[pl.ds(si * 16, 16)], sg0)
        drain_rows(_NSL * 16, rows0, sg0)
        accum16(rows0, outb0)
        pltpu.sync_copy(outb0.at[pl.ds(0, 16)],
                        out_hbm.at[b, pl.ds(off_e + 448, 16)])


def _gather_accumulate(cidx, mp, r):
    mesh = plsc.VectorSubcoreMesh(core_axis_name="c", subcore_axis_name="s")
    return pl.kernel(
        _k3_body,
        out_type=jax.ShapeDtypeStruct((B, NF, DIM), jnp.float32),
        mesh=mesh,
        compiler_params=pltpu.CompilerParams(needs_layout_passes=False,
                                             use_tc_tiling_on_sc=False),
        scratch_types=[
            pltpu.VMEM((NS, 512), jnp.int32),
            pltpu.VMEM((_NSL * 32, 64), jnp.float32),
            pltpu.VMEM((_NSL * 32, 64), jnp.float32),
            pltpu.VMEM((_NSL * 32, 64), jnp.float32),
            pltpu.VMEM((32, DIM), jnp.float32),
            pltpu.VMEM((32, DIM), jnp.float32),
            pltpu.VMEM((32, DIM), jnp.float32),
            pltpu.VMEM((DIM,), jnp.float32),
            pltpu.SemaphoreType.DMA,
            pltpu.SemaphoreType.DMA,
            pltpu.SemaphoreType.DMA,
            pltpu.SemaphoreType.DMA,
            pltpu.SemaphoreType.DMA,
            pltpu.SemaphoreType.DMA,
        ],
    )(cidx, mp, r)


def kernel(vertices, faces, in_theta, in_phi, in_freq, coor_embed,
           angle_embed, area_embed, emnoangle_embed, emangle_embed,
           emfreq_embed, normal_embed, W, bias):
    tables = jnp.stack(
        [coor_embed] * 9 + [angle_embed] * 3 + [area_embed, emnoangle_embed]
        + [emangle_embed] * 3 + [emfreq_embed] + [normal_embed] * 3, axis=0)
    th2 = in_theta.reshape(B, 1)
    ph2 = in_phi.reshape(B, 1)
    fr2 = in_freq.reshape(B, 1)
    m, r = _fold_tables(tables, W.reshape(NS, DE, DIM), bias.reshape(1, DIM),
                        th2, ph2, fr2)
    faces_flat = jnp.pad(faces, ((0, 0), (0, 176), (0, 0))).reshape(B, -1)
    fc = _gather_face_coords(vertices.reshape(B, NV * 3), faces_flat)
    cidx4 = _discretize_faces(fc.reshape(B, 9, G4, 256), th2, ph2, fr2)
    mb = m.astype(jnp.bfloat16)
    mp = lax.bitcast_convert_type(
        jnp.stack([mb[:, :64], mb[:, 64:]], axis=-1), jnp.float32)
    out = _gather_accumulate(cidx4.reshape(B, NS, NF_PAD), mp, r)
    return out
